# baseline (speedup 1.0000x reference)
import jax
import jax.numpy as jnp
from jax import lax
from jax.experimental import pallas as pl
from jax.experimental.pallas import tpu as pltpu
from jax.experimental.pallas import tpu_sc as plsc

EMB = 32
NC = 2
NS = 16
NW = NC * NS
CB = 64


def _gather_call(B, L, idx, table):
    b_per_w = B // NW
    n_chunks = b_per_w // CB
    HB = CB // 2

    LP = (L + 7) // 8 * 8
    MP = 128
    GL = LP
    rows_per_ch = CB * GL

    mesh = plsc.VectorSubcoreMesh(
        core_axis_name="c", subcore_axis_name="s", num_cores=NC,
        num_subcores=NS)

    @pl.kernel(
        out_type=jax.ShapeDtypeStruct((B, LP, MP), jnp.float32),
        mesh=mesh,
        compiler_params=pltpu.CompilerParams(use_tc_tiling_on_sc=False),
        scratch_types=[
            pltpu.VMEM((CB, MP), jnp.int32),
            pltpu.VMEM((rows_per_ch, EMB), jnp.float32),
            pltpu.SemaphoreType.DMA,
            pltpu.SemaphoreType.DMA,
            pltpu.SemaphoreType.DMA,
        ],
    )
    def k(idx_hbm, table_hbm, out_hbm, idx_v, rows_v, sg, soa, sob):
        wid = lax.axis_index("s") * NC + lax.axis_index("c")
        batch0 = wid * b_per_w

        def fire_gathers(c, h):
            def one(i):
                pltpu.async_copy(
                    table_hbm.at[idx_v.at[h * HB + i, pl.ds(0, GL)]],
                    rows_v.at[pl.ds((h * HB + i) * GL, GL), :],
                    sg,
                )
            pl.loop(0, HB)(one)

        def wait_gathers():
            def one(i):
                pltpu.make_async_copy(
                    table_hbm.at[idx_v.at[0, pl.ds(0, GL)]],
                    rows_v.at[pl.ds(0, GL), :],
                    sg,
                ).wait()
            pl.loop(0, HB)(one)

        def fire_outs(c, h, sem):
            def one(bb):
                pltpu.async_copy(
                    rows_v.at[pl.ds((h * HB + bb) * GL, L), :],
                    out_hbm.at[batch0 + c * CB + h * HB + bb,
                               pl.ds(0, L), pl.ds(0, EMB)],
                    sem,
                )
            pl.loop(0, HB)(one)

        def wait_outs(sem):
            def one(bb):
                pltpu.make_async_copy(
                    rows_v.at[pl.ds(0, L), :],
                    out_hbm.at[batch0, pl.ds(0, L), pl.ds(0, EMB)],
                    sem,
                ).wait()
            pl.loop(0, HB)(one)

        def chunk(c):
            pltpu.sync_copy(idx_hbm.at[pl.ds(batch0 + c * CB, CB), :], idx_v)
            def drain_prev():
                wait_outs(soa)
                wait_outs(sob)
            pl.when(c > 0)(drain_prev)
            fire_gathers(c, 0)
            wait_gathers()
            fire_outs(c, 0, soa)
            fire_gathers(c, 1)
            wait_gathers()
            fire_outs(c, 1, sob)

        pl.loop(0, n_chunks)(chunk)
        wait_outs(soa)
        wait_outs(sob)

    out_padded = k(idx, table)
    return out_padded[:, :L, :EMB]


def kernel(inputs, table):
    B, L = inputs.shape
    if inputs.dtype != jnp.int32:
        inputs = inputs.astype(jnp.int32)
    inputs_p = jnp.pad(inputs, ((0, 0), (0, 128 - L)))
    return _gather_call(B, L, inputs_p, table)


# speedup vs baseline: 2.5258x; 2.5258x over previous
import jax
import jax.numpy as jnp
from jax import lax
from jax.experimental import pallas as pl
from jax.experimental.pallas import tpu as pltpu
from jax.experimental.pallas import tpu_sc as plsc

EMB = 32
NC = 2
NS = 16
NW = NC * NS
SUB = 128
CB = 64


def _gather_call(B, L, idx2d, table):
    n_rows = B * L
    b_per_w = B // NW
    rows_per_w = b_per_w * L
    sub_per_w = rows_per_w // SUB
    rows_per_ch = CB * L
    sub_per_ch = rows_per_ch // SUB
    n_chunks = b_per_w // CB
    half_rows = rows_per_ch // 2
    assert half_rows == (CB // 2) * L
    sub_a = (half_rows + SUB - 1) // SUB

    LP = (L + 7) // 8 * 8
    MP = 128

    mesh = plsc.VectorSubcoreMesh(
        core_axis_name="c", subcore_axis_name="s", num_cores=NC,
        num_subcores=NS)

    @pl.kernel(
        out_type=jax.ShapeDtypeStruct((B, LP, MP), jnp.float32),
        mesh=mesh,
        compiler_params=pltpu.CompilerParams(use_tc_tiling_on_sc=False),
        scratch_types=[
            pltpu.VMEM((sub_per_w, SUB), jnp.int32),
            pltpu.VMEM((rows_per_ch, EMB), jnp.float32),
            pltpu.SemaphoreType.DMA,
            pltpu.SemaphoreType.DMA,
            pltpu.SemaphoreType.DMA,
        ],
    )
    def k(idx_hbm, table_hbm, out_hbm, idx_v, rows_v, sg, soa, sob):
        wid = lax.axis_index("s") * NC + lax.axis_index("c")
        batch0 = wid * b_per_w
        row0 = wid * sub_per_w

        pltpu.sync_copy(idx_hbm.at[pl.ds(row0, sub_per_w), :], idx_v)

        def fire_gathers(c, j0, j1):
            for j in range(j0, j1):
                pltpu.async_copy(
                    table_hbm.at[idx_v.at[c * sub_per_ch + j]],
                    rows_v.at[pl.ds(j * SUB, SUB), :],
                    sg,
                )

        def wait_gathers(j0, j1):
            for j in range(j0, j1):
                pltpu.make_async_copy(
                    table_hbm.at[idx_v.at[j]],
                    rows_v.at[pl.ds(j * SUB, SUB), :],
                    sg,
                ).wait()

        def fire_outs(c, h, sem):
            def one(bb):
                pltpu.async_copy(
                    rows_v.at[pl.ds(h * half_rows + bb * L, L), :],
                    out_hbm.at[batch0 + c * CB + h * (CB // 2) + bb,
                               pl.ds(0, L), pl.ds(0, EMB)],
                    sem,
                )
            pl.loop(0, CB // 2)(one)

        def wait_outs(sem):
            def one(bb):
                pltpu.make_async_copy(
                    rows_v.at[pl.ds(0, L), :],
                    out_hbm.at[batch0, pl.ds(0, L), pl.ds(0, EMB)],
                    sem,
                ).wait()
            pl.loop(0, CB // 2)(one)

        def chunk(c):
            def drain_prev():
                wait_outs(soa)
                wait_outs(sob)
            pl.when(c > 0)(drain_prev)
            fire_gathers(c, 0, sub_a)
            wait_gathers(0, sub_a)
            fire_outs(c, 0, soa)
            fire_gathers(c, sub_a, sub_per_ch)
            wait_gathers(sub_a, sub_per_ch)
            fire_outs(c, 1, sob)

        pl.loop(0, n_chunks)(chunk)
        wait_outs(soa)
        wait_outs(sob)

    out_padded = k(idx2d, table)
    return out_padded[:, :L, :EMB]


def kernel(inputs, table):
    B, L = inputs.shape
    idx = inputs.reshape(-1)
    if idx.dtype != jnp.int32:
        idx = idx.astype(jnp.int32)
    idx2d = idx.reshape((B * L) // SUB, SUB)
    return _gather_call(B, L, idx2d, table)


# speedup vs baseline: 2.5260x; 1.0001x over previous
import jax
import jax.numpy as jnp
from jax import lax
from jax.experimental import pallas as pl
from jax.experimental.pallas import tpu as pltpu
from jax.experimental.pallas import tpu_sc as plsc

EMB = 32
NC = 2
NS = 16
NW = NC * NS
SUB = 128
CB = 64


def _gather_call(B, L, idx2d, table):
    n_rows = B * L
    b_per_w = B // NW
    rows_per_w = b_per_w * L
    sub_per_w = rows_per_w // SUB
    rows_per_ch = CB * L
    sub_per_ch = rows_per_ch // SUB
    n_chunks = b_per_w // CB
    half_rows = rows_per_ch // 2
    assert half_rows == (CB // 2) * L
    sub_a = (half_rows + SUB - 1) // SUB

    LP = (L + 7) // 8 * 8
    MP = 128

    mesh = plsc.VectorSubcoreMesh(
        core_axis_name="c", subcore_axis_name="s", num_cores=NC,
        num_subcores=NS)

    @pl.kernel(
        out_type=jax.ShapeDtypeStruct((B, LP, MP), jnp.float32),
        mesh=mesh,
        compiler_params=pltpu.CompilerParams(use_tc_tiling_on_sc=False),
        scratch_types=[
            pltpu.VMEM((sub_per_w, SUB), jnp.int32),
            pltpu.VMEM((rows_per_ch, EMB), jnp.float32),
            pltpu.SemaphoreType.DMA,
            pltpu.SemaphoreType.DMA,
            pltpu.SemaphoreType.DMA,
        ],
    )
    def k(table_hbm, idx_hbm, out_hbm, idx_v, rows_v, sg, soa, sob):
        wid = lax.axis_index("s") * NC + lax.axis_index("c")
        batch0 = wid * b_per_w
        row0 = wid * sub_per_w

        pltpu.sync_copy(idx_hbm.at[pl.ds(row0, sub_per_w), :], idx_v)

        def fire_gathers(c, j0, j1):
            for j in range(j0, j1):
                pltpu.async_copy(
                    table_hbm.at[idx_v.at[c * sub_per_ch + j]],
                    rows_v.at[pl.ds(j * SUB, SUB), :],
                    sg,
                )

        def wait_gathers(j0, j1):
            for j in range(j0, j1):
                pltpu.make_async_copy(
                    table_hbm.at[idx_v.at[j]],
                    rows_v.at[pl.ds(j * SUB, SUB), :],
                    sg,
                ).wait()

        def fire_outs(c, h, sem):
            def one(bb):
                pltpu.async_copy(
                    rows_v.at[pl.ds(h * half_rows + bb * L, L), :],
                    out_hbm.at[batch0 + c * CB + h * (CB // 2) + bb,
                               pl.ds(0, L), pl.ds(0, EMB)],
                    sem,
                )
            pl.loop(0, CB // 2)(one)

        def wait_outs(sem):
            def one(bb):
                pltpu.make_async_copy(
                    rows_v.at[pl.ds(0, L), :],
                    out_hbm.at[batch0, pl.ds(0, L), pl.ds(0, EMB)],
                    sem,
                ).wait()
            pl.loop(0, CB // 2)(one)

        def chunk(c):
            def drain_prev():
                wait_outs(soa)
                wait_outs(sob)
            pl.when(c > 0)(drain_prev)
            fire_gathers(c, 0, sub_a)
            wait_gathers(0, sub_a)
            fire_outs(c, 0, soa)
            fire_gathers(c, sub_a, sub_per_ch)
            wait_gathers(sub_a, sub_per_ch)
            fire_outs(c, 1, sob)

        pl.loop(0, n_chunks)(chunk)
        wait_outs(soa)
        wait_outs(sob)

    out_padded = k(table, idx2d)
    return out_padded[:, :L, :EMB]


def kernel(inputs, table):
    B, L = inputs.shape
    idx = inputs.reshape(-1)
    if idx.dtype != jnp.int32:
        idx = idx.astype(jnp.int32)
    idx2d = idx.reshape((B * L) // SUB, SUB)
    return _gather_call(B, L, idx2d, table)
